# Initial kernel scaffold; baseline (speedup 1.0000x reference)
import jax
import jax.numpy as jnp
from jax.experimental import pallas as pl


def kernel(x, dyn, kernel_dyn_weights, kernel_weights, bias, neighbor_index):
    raise NotImplementedError("write your pallas kernel here")


# speedup vs baseline: 14.2260x; 14.2260x over previous
import numpy as np
import jax
import jax.numpy as jnp
from jax.experimental import pallas as pl
from jax.experimental.pallas import tpu as pltpu

_KS = 7
_TAPS = ((-1, -1), (-1, 0), (0, -1), (0, 0), (0, 1), (1, 0), (1, 1))


def _hex_layout(num_blocks):
    radius = int(round((3 + np.sqrt(12 * num_blocks - 3)) / 6))
    assert 3 * radius * radius - 3 * radius + 1 == num_blocks
    r = radius - 1
    gj = (2 * r + 3 + 7) // 8 * 8
    rows = []
    start = 0
    for gi, i in enumerate(range(-r, r + 1)):
        jmin = max(-r, i - r)
        jmax = min(r, i + r)
        ln = jmax - jmin + 1
        dest = (gi + 2) * gj + (jmin + r + 1)
        rows.append((start, ln, dest))
        start += ln
    assert start == num_blocks
    gi_total = len(rows) + 4
    return gj, gi_total, rows


def _body(dyn_ref, kdw_ref, kw_ref, bias_ref, x_hbm, out_hbm,
          stage, xg, w_scr, sem_in, sem_out,
          *, n, k_taps, d_dim, gj, rows, rpc, n_chunks):
    b = pl.program_id(0)

    @pl.when(b == 0)
    def _zero():
        xg[...] = jnp.zeros_like(xg)

    cp_in = pltpu.make_async_copy(x_hbm.at[b], stage.at[pl.ds(0, n)], sem_in)
    cp_in.start()

    for k in range(k_taps):
        acc = kw_ref[k]
        for d in range(d_dim):
            acc = acc + dyn_ref[0, k, d] * kdw_ref[k, d]
        w_scr[k] = acc

    cp_in.wait()

    for (rs, ln, dest) in rows:
        xg[pl.ds(dest, ln), :] = stage[pl.ds(rs, ln), :]

    base = 2 * gj
    chunk = rpc * gj
    offs = tuple(di * gj + dj for (di, dj) in _TAPS)
    for c in range(n_chunks):
        acc = None
        for k in range(k_taps):
            a = xg[pl.ds(base + c * chunk + offs[k], chunk), :]
            p = jax.lax.dot_general(
                a, w_scr[k], (((1,), (0,)), ((), ())),
                preferred_element_type=jnp.float32,
                precision=jax.lax.Precision.HIGHEST)
            acc = p if acc is None else acc + p
        acc = acc + bias_ref[...]
        for gi in range(c * rpc, (c + 1) * rpc):
            rs, ln, dest = rows[gi]
            local = dest - base - c * chunk
            stage[pl.ds(rs, ln), :] = acc[local:local + ln, :]

    cp_out = pltpu.make_async_copy(stage.at[pl.ds(0, n)], out_hbm.at[b],
                                   sem_out)
    cp_out.start()
    cp_out.wait()


def kernel(x, dyn, kernel_dyn_weights, kernel_weights, bias, neighbor_index):
    bsz, n, f = x.shape
    k_taps, d_dim = dyn.shape[1], dyn.shape[2]
    o_dim = kernel_weights.shape[-1]
    gj, gi_total, rows = _hex_layout(n)
    l_flat = gi_total * gj
    n_rows = gi_total - 4
    rpc = 1
    for cand in range(12, 0, -1):
        if n_rows % cand == 0:
            rpc = cand
            break
    n_chunks = n_rows // rpc
    n_pad = (n + 7) // 8 * 8

    bias2 = bias.reshape(1, o_dim)

    body = lambda *refs: _body(
        *refs, n=n, k_taps=k_taps, d_dim=d_dim, gj=gj, rows=tuple(rows),
        rpc=rpc, n_chunks=n_chunks)

    out = pl.pallas_call(
        body,
        grid=(bsz,),
        in_specs=[
            pl.BlockSpec((1, k_taps, d_dim), lambda b: (b, 0, 0),
                         memory_space=pltpu.SMEM),
            pl.BlockSpec((k_taps, d_dim, f, o_dim), lambda b: (0, 0, 0, 0)),
            pl.BlockSpec((k_taps, f, o_dim), lambda b: (0, 0, 0)),
            pl.BlockSpec((1, o_dim), lambda b: (0, 0)),
            pl.BlockSpec(memory_space=pl.ANY),
        ],
        out_specs=pl.BlockSpec(memory_space=pl.ANY),
        out_shape=jax.ShapeDtypeStruct((bsz, n, o_dim), jnp.float32),
        scratch_shapes=[
            pltpu.VMEM((n_pad, f), jnp.float32),
            pltpu.VMEM((l_flat, f), jnp.float32),
            pltpu.VMEM((k_taps, f, o_dim), jnp.float32),
            pltpu.SemaphoreType.DMA,
            pltpu.SemaphoreType.DMA,
        ],
        compiler_params=pltpu.CompilerParams(
            dimension_semantics=("arbitrary",),
            vmem_limit_bytes=65472 * 1024),
    )(dyn, kernel_dyn_weights, kernel_weights, bias2, x)
    return out


# speedup vs baseline: 83.2414x; 5.8514x over previous
import numpy as np
import jax
import jax.numpy as jnp
from jax.experimental import pallas as pl
from jax.experimental.pallas import tpu as pltpu

_KS = 7
_TAPS = ((-1, -1), (-1, 0), (0, -1), (0, 0), (0, 1), (1, 0), (1, 1))


def _hex_layout(num_blocks):
    radius = int(round((3 + np.sqrt(12 * num_blocks - 3)) / 6))
    assert 3 * radius * radius - 3 * radius + 1 == num_blocks
    r = radius - 1
    gj = (2 * r + 3 + 7) // 8 * 8
    rows = []
    start = 0
    for gi, i in enumerate(range(-r, r + 1)):
        jmin = max(-r, i - r)
        jmax = min(r, i + r)
        ln = jmax - jmin + 1
        dest = (gi + 2) * gj + (jmin + r + 1)
        rows.append((start, ln, dest))
        start += ln
    assert start == num_blocks
    gi_total = len(rows) + 4
    return gj, gi_total, rows


def _body(dyn_ref, kdw_ref, kw_ref, bias_ref, x_hbm, out_hbm,
          stage, xg, w_scr, sem_in, sem_out,
          *, n, k_taps, d_dim, gj, rows, rpc, n_chunks):
    b = pl.program_id(0)

    @pl.when(b == 0)
    def _zero():
        xg[...] = jnp.zeros_like(xg)

    cp_in = pltpu.make_async_copy(x_hbm.at[b], stage.at[pl.ds(0, n)], sem_in)
    cp_in.start()

    for k in range(k_taps):
        acc = kw_ref[k]
        for d in range(d_dim):
            acc = acc + dyn_ref[0, k, d] * kdw_ref[k, d]
        w_scr[pl.ds(k * acc.shape[0], acc.shape[0]), :] = acc

    cp_in.wait()

    for (rs, ln, dest) in rows:
        xg[pl.ds(dest, ln), :] = stage[pl.ds(rs, ln), :]

    base = 2 * gj
    chunk = rpc * gj
    offs = tuple(di * gj + dj for (di, dj) in _TAPS)
    for c in range(n_chunks):
        a = jnp.concatenate(
            [xg[pl.ds(base + c * chunk + offs[k], chunk), :]
             for k in range(k_taps)], axis=1)
        acc = jax.lax.dot_general(
            a, w_scr[...], (((1,), (0,)), ((), ())),
            preferred_element_type=jnp.float32)
        acc = acc + bias_ref[...]
        for gi in range(c * rpc, (c + 1) * rpc):
            rs, ln, dest = rows[gi]
            local = dest - base - c * chunk
            stage[pl.ds(rs, ln), :] = acc[local:local + ln, :]

    cp_out = pltpu.make_async_copy(stage.at[pl.ds(0, n)], out_hbm.at[b],
                                   sem_out)
    cp_out.start()
    cp_out.wait()


def kernel(x, dyn, kernel_dyn_weights, kernel_weights, bias, neighbor_index):
    bsz, n, f = x.shape
    k_taps, d_dim = dyn.shape[1], dyn.shape[2]
    o_dim = kernel_weights.shape[-1]
    gj, gi_total, rows = _hex_layout(n)
    l_flat = gi_total * gj
    n_rows = gi_total - 4
    rpc = 1
    for cand in range(12, 0, -1):
        if n_rows % cand == 0:
            rpc = cand
            break
    n_chunks = n_rows // rpc
    n_pad = (n + 7) // 8 * 8

    bias2 = bias.reshape(1, o_dim)

    body = lambda *refs: _body(
        *refs, n=n, k_taps=k_taps, d_dim=d_dim, gj=gj, rows=tuple(rows),
        rpc=rpc, n_chunks=n_chunks)

    out = pl.pallas_call(
        body,
        grid=(bsz,),
        in_specs=[
            pl.BlockSpec((1, k_taps, d_dim), lambda b: (b, 0, 0),
                         memory_space=pltpu.SMEM),
            pl.BlockSpec((k_taps, d_dim, f, o_dim), lambda b: (0, 0, 0, 0)),
            pl.BlockSpec((k_taps, f, o_dim), lambda b: (0, 0, 0)),
            pl.BlockSpec((1, o_dim), lambda b: (0, 0)),
            pl.BlockSpec(memory_space=pl.ANY),
        ],
        out_specs=pl.BlockSpec(memory_space=pl.ANY),
        out_shape=jax.ShapeDtypeStruct((bsz, n, o_dim), jnp.float32),
        scratch_shapes=[
            pltpu.VMEM((n_pad, f), jnp.float32),
            pltpu.VMEM((l_flat, f), jnp.float32),
            pltpu.VMEM((k_taps * f, o_dim), jnp.float32),
            pltpu.SemaphoreType.DMA,
            pltpu.SemaphoreType.DMA,
        ],
        compiler_params=pltpu.CompilerParams(
            dimension_semantics=("arbitrary",),
            vmem_limit_bytes=65472 * 1024),
    )(dyn, kernel_dyn_weights, kernel_weights, bias2, x)
    return out
